# baseline (speedup 1.0000x reference)
import functools
import math

import jax
import jax.numpy as jnp
from jax import lax
from jax.experimental import pallas as pl
from jax.experimental.pallas import tpu as pltpu
from jax.experimental.pallas import tpu_sc as plsc

D_MODEL = 64
SCALE = math.sqrt(D_MODEL)
LANES = 16
NC, NS = 2, 16
NW = NC * NS
BB = 128


def _make_kernel(bsz: int, seq: int, vocab2: int):
    assert bsz == NW * BB and seq % 2 == 0
    mesh = plsc.VectorSubcoreMesh(core_axis_name="c", subcore_axis_name="s")

    @functools.partial(
        pl.kernel,
        out_type=jax.ShapeDtypeStruct((seq, D_MODEL, bsz), jnp.float32),
        mesh=mesh,
        scratch_types=[
            pltpu.VMEM((BB, seq), jnp.int32),
            pltpu.VMEM((seq, BB), jnp.int32),
            pltpu.VMEM((2, BB, 128), jnp.float32),
            pltpu.VMEM((2, D_MODEL, BB), jnp.float32),
            pltpu.SemaphoreType.DMA,
            pltpu.SemaphoreType.DMA,
            pltpu.SemaphoreType.DMA,
            pltpu.SemaphoreType.DMA,
        ],
        compiler_params=pltpu.CompilerParams(
            use_tc_tiling_on_sc=True, needs_layout_passes=False
        ),
    )
    def emb_kernel(x_hbm, tab2_hbm, out_hbm, xv, idxt, gbuf, obuf,
                   gsem0, gsem1, osem0, osem1):
        wid = lax.axis_index("s") * NC + lax.axis_index("c")
        b0 = wid * BB
        gsems = (gsem0, gsem1)
        osems = (osem0, osem1)
        iota = lax.iota(jnp.int32, LANES)

        pltpu.sync_copy(x_hbm.at[pl.ds(b0, BB)], xv)

        @pl.loop(0, seq)
        def _build(l):
            lvec = jnp.full((LANES,), 0, jnp.int32) + l
            for k in range(BB // LANES):
                rows = iota + (k * LANES)
                v = plsc.load_gather(xv, [rows, lvec])
                idxt[l, pl.ds(k * LANES, LANES)] = v >> 1

        def fire(l, slot):
            pltpu.async_copy(
                tab2_hbm.at[idxt.at[l]], gbuf.at[slot], gsems[slot]
            )

        def drain(l, slot):
            pltpu.make_async_copy(
                tab2_hbm.at[idxt.at[l]], gbuf.at[slot], gsems[slot]
            ).wait()

        def owait(l, slot):
            pltpu.make_async_copy(
                obuf.at[slot],
                out_hbm.at[l, :, pl.ds(b0, BB)],
                osems[slot],
            ).wait()

        fire(0, 0)

        @pl.loop(0, seq, step=2)
        def l_loop(g):
            for s in range(2):
                l = g + s

                @pl.when(l + 1 < seq)
                def _start_next():
                    fire(l + 1, 1 - s)

                drain(l, s)

                @pl.when(l >= 2)
                def _wait_prev_out():
                    owait(l - 2, s)

                g2 = gbuf.at[s]
                for k in range(BB // LANES):
                    rows = iota + (k * LANES)
                    lvec = jnp.full((LANES,), 0, jnp.int32) + l
                    par = plsc.load_gather(xv, [rows, lvec]) & 1
                    col0 = par * D_MODEL
                    for d in range(D_MODEL):
                        v = plsc.load_gather(g2, [rows, col0 + d])
                        obuf[s, d, pl.ds(k * LANES, LANES)] = v * SCALE

                pltpu.async_copy(
                    obuf.at[s],
                    out_hbm.at[l, :, pl.ds(b0, BB)],
                    osems[s],
                )

        owait(seq - 2, 0)
        owait(seq - 1, 1)

    return emb_kernel


def kernel(x, table):
    b, l = x.shape
    v, d = table.shape
    table2 = table.reshape(v // 2, 2 * d)
    out_t = _make_kernel(b, l, v // 2)(x.astype(jnp.int32), table2)
    return out_t.transpose(2, 0, 1)
